# baseline (speedup 1.0000x reference)
import functools

import jax
import jax.numpy as jnp
from jax import lax
from jax.experimental import pallas as pl
from jax.experimental.pallas import tpu as pltpu
from jax.experimental.pallas import tpu_sc as plsc

B, L, V, D = 1024, 200, 100000, 128
NC, NS, LANES = 2, 16, 16
NW = NC * NS
SEQ_PER_W = B // NW
G = 4
PH = 5
RPC = L // PH
GROUPS = SEQ_PER_W // G
STEPS = PH * GROUPS
NBUF = 5


def _sc_body(x_hbm, tok_hbm, pos_hbm, out_hbm, idx_all,
             b0, b1, b2, b3, b4,
             ps0, ps1, ps2, ps3, ps4,
             gs0, gs1, gs2, gs3, gs4,
             os0, os1, os2, os3, os4):
    bufs = (b0, b1, b2, b3, b4)
    psems = (ps0, ps1, ps2, ps3, ps4)
    gsems = (gs0, gs1, gs2, gs3, gs4)
    osems = (os0, os1, os2, os3, os4)

    wid = lax.axis_index("s") * NC + lax.axis_index("c")
    seq0 = wid * SEQ_PER_W

    pltpu.sync_copy(x_hbm.at[pl.ds(seq0 * L, SEQ_PER_W * L)], idx_all)

    def start_prefills(k):
        c = k // GROUPS
        bset = bufs[k % NBUF]
        sem = psems[k % NBUF]
        return tuple(
            pltpu.async_copy(
                pos_hbm.at[pl.ds(c * RPC, RPC)], bset.at[j], sem)
            for j in range(G)
        )

    def start_gadds(k):
        c, g = k // GROUPS, k % GROUPS
        bset = bufs[k % NBUF]
        sem = gsems[k % NBUF]
        return tuple(
            pltpu.async_copy(
                tok_hbm.at[idx_all.at[pl.ds((g * G + j) * L + c * RPC, RPC)]],
                bset.at[j], sem, add=True)
            for j in range(G)
        )

    def start_outs(k):
        c, g = k // GROUPS, k % GROUPS
        bset = bufs[k % NBUF]
        sem = osems[k % NBUF]
        return tuple(
            pltpu.async_copy(
                bset.at[j],
                out_hbm.at[pl.ds((seq0 + g * G + j) * L + c * RPC, RPC)],
                sem)
            for j in range(G)
        )

    prefills = {k: start_prefills(k) for k in range(3)}
    for cp in prefills.pop(0):
        cp.wait()
    gadds = {0: start_gadds(0)}
    outs = {}

    for k in range(STEPS):
        for cp in gadds.pop(k):
            cp.wait()
        outs[k] = start_outs(k)

        t1 = k + 1
        if t1 < STEPS:
            for cp in prefills.pop(t1):
                cp.wait()
            gadds[t1] = start_gadds(t1)

        t2 = k + 3
        if t2 < STEPS:
            if t2 >= NBUF:
                for cp in outs.pop(t2 - NBUF):
                    cp.wait()
            prefills[t2] = start_prefills(t2)

    for k in sorted(outs):
        for cp in outs.pop(k):
            cp.wait()


@jax.jit
def _run(x, token_table, pos_table):
    mesh = plsc.VectorSubcoreMesh(core_axis_name="c", subcore_axis_name="s")
    buf = pltpu.VMEM((G, RPC, D), jnp.float32)
    sem = pltpu.SemaphoreType.DMA
    kfn = functools.partial(
        pl.kernel,
        mesh=mesh,
        out_type=jax.ShapeDtypeStruct((B * L, D), jnp.float32),
        scratch_types=[pltpu.VMEM((SEQ_PER_W * L,), jnp.int32)]
        + [buf] * NBUF + [sem] * (3 * NBUF),
    )(_sc_body)
    return kfn(x, token_table, pos_table)


def kernel(x, token_table, pos_table):
    out = _run(x.astype(jnp.int32).reshape(B * L), token_table, pos_table)
    return out.reshape(B, L, D)


# speedup vs baseline: 3.9644x; 3.9644x over previous
import functools

import jax
import jax.numpy as jnp
from jax import lax
from jax.experimental import pallas as pl
from jax.experimental.pallas import tpu as pltpu
from jax.experimental.pallas import tpu_sc as plsc

B, L, V, D = 1024, 200, 100000, 128
NC, NS, LANES = 2, 16, 16
NW = NC * NS
SEQ_PER_W = B // NW
VECS_PER_ROW = D // LANES
G = 4
PH = 5
RPC = L // PH
GROUPS = SEQ_PER_W // G
STEPS = PH * GROUPS
NBUF = 4
BROWS = G * RPC


def _sc_body(x_hbm, tok_hbm, pos_hbm, out_hbm,
             idx_all, pos_v, b0, b1, b2, b3,
             gs0, gs1, gs2, gs3, os0, os1, os2, os3, psem):
    bufs = (b0, b1, b2, b3)
    gsems = (gs0, gs1, gs2, gs3)
    osems = (os0, os1, os2, os3)

    wid = lax.axis_index("s") * NC + lax.axis_index("c")
    seq0 = wid * SEQ_PER_W

    pos_cp = pltpu.async_copy(pos_hbm, pos_v, psem)
    pltpu.sync_copy(x_hbm.at[pl.ds(seq0 * L, SEQ_PER_W * L)], idx_all)

    def start_gathers(k):
        c, g = k // GROUPS, k % GROUPS
        bset = bufs[k % NBUF]
        sem = gsems[k % NBUF]
        for j in range(G):
            pltpu.async_copy(
                tok_hbm.at[idx_all.at[pl.ds((g * G + j) * L + c * RPC, RPC)]],
                bset.at[pl.ds(j * RPC, RPC)], sem)

    def wait_gathers(k):
        bset = bufs[k % NBUF]
        pltpu.make_async_copy(
            tok_hbm.at[pl.ds(0, BROWS)], bset, gsems[k % NBUF]).wait()

    def start_outs(k):
        c, g = k // GROUPS, k % GROUPS
        bset = bufs[k % NBUF]
        sem = osems[k % NBUF]
        for j in range(G):
            pltpu.async_copy(
                bset.at[pl.ds(j * RPC, RPC)],
                out_hbm.at[pl.ds((seq0 + g * G + j) * L + c * RPC, RPC)],
                sem)

    def wait_outs(k):
        bset = bufs[k % NBUF]
        pltpu.make_async_copy(
            bset, out_hbm.at[pl.ds(0, BROWS)], osems[k % NBUF]).wait()

    start_gathers(0)
    start_gathers(1)

    pos_cp.wait()
    for k in range(STEPS):
        c = k // GROUPS
        bset = bufs[k % NBUF]
        wait_gathers(k)

        @plsc.parallel_loop(0, RPC, step=1, unroll=2)
        def add_body(r, bset=bset, c=c):
            pvs = [
                pos_v[c * RPC + r, pl.ds(ci * LANES, LANES)]
                for ci in range(VECS_PER_ROW)
            ]
            for j in range(G):
                for ci in range(VECS_PER_ROW):
                    plsc.addupdate(
                        bset.at[j * RPC + r, pl.ds(ci * LANES, LANES)],
                        pvs[ci])

        start_outs(k)

        t = k + 2
        if t < STEPS:
            if t >= NBUF:
                wait_outs(t - NBUF)
            start_gathers(t)

    for k in range(STEPS - NBUF, STEPS):
        wait_outs(k)


@jax.jit
def _run(x, token_table, pos_table):
    mesh = plsc.VectorSubcoreMesh(core_axis_name="c", subcore_axis_name="s")
    buf = pltpu.VMEM((BROWS, D), jnp.float32)
    sem = pltpu.SemaphoreType.DMA
    kfn = functools.partial(
        pl.kernel,
        mesh=mesh,
        out_type=jax.ShapeDtypeStruct((B * L, D), jnp.float32),
        scratch_types=[
            pltpu.VMEM((SEQ_PER_W * L,), jnp.int32),
            pltpu.VMEM((L, D), jnp.float32),
        ] + [buf] * NBUF + [sem] * (2 * NBUF + 1),
    )(_sc_body)
    return kfn(x, token_table, pos_table)


def kernel(x, token_table, pos_table):
    out = _run(x.astype(jnp.int32).reshape(B * L), token_table, pos_table)
    return out.reshape(B, L, D)
